# baseline (speedup 1.0000x reference)
import functools

import jax
import jax.numpy as jnp
from jax import lax
from jax.experimental import pallas as pl
from jax.experimental.pallas import tpu as pltpu
from jax.experimental.pallas import tpu_sc as plsc

_NC = 2
_NS = 16
_NW = _NC * _NS
_CHUNK = 128
_HP = 16
_BLKN = 4096


def _proj_body(t_ref, w_ref, b_ref, o_ref):
    o_ref[...] = lax.dot_general(
        t_ref[...], w_ref[...], (((0,), (0,)), ((), ())),
        preferred_element_type=jnp.float32) + b_ref[...]


def _project(tab_t, w, b, nsteps):
    K = tab_t.shape[0]
    return pl.pallas_call(
        _proj_body,
        grid=(nsteps,),
        in_specs=[
            pl.BlockSpec((K, _BLKN), lambda i: (0, i)),
            pl.BlockSpec((K, _HP), lambda i: (0, 0)),
            pl.BlockSpec((1, _HP), lambda i: (0, 0)),
        ],
        out_specs=pl.BlockSpec((_BLKN, _HP), lambda i: (i, 0)),
        out_shape=jax.ShapeDtypeStruct((nsteps * _BLKN, _HP), jnp.float32),
    )(tab_t, w, b)


def _make_sc_zdot(B, H):
    assert B % (_NW * _CHUNK) == 0
    b_per_w = B // _NW
    chunks = b_per_w // _CHUNK
    mesh = plsc.VectorSubcoreMesh(core_axis_name="c", subcore_axis_name="s")

    @functools.partial(
        pl.kernel,
        mesh=mesh,
        out_type=jax.ShapeDtypeStruct((B,), jnp.float32),
        scratch_types=[
            pltpu.VMEM((chunks, _CHUNK), jnp.int32),
            pltpu.VMEM((chunks, _CHUNK), jnp.int32),
            pltpu.VMEM((2, _CHUNK, _HP), jnp.float32),
            pltpu.VMEM((2, _CHUNK, _HP), jnp.float32),
            pltpu.VMEM((b_per_w,), jnp.float32),
            pltpu.SemaphoreType.DMA,
            pltpu.SemaphoreType.DMA,
            pltpu.SemaphoreType.DMA,
            pltpu.SemaphoreType.DMA,
        ],
        compiler_params=pltpu.CompilerParams(needs_layout_passes=False,
                                             use_tc_tiling_on_sc=False),
    )
    def sc_zdot(uidx_hbm, midx_hbm, utab_hbm, mtab_hbm,
                z_out, uidx_v, midx_v, ubuf, mbuf, z_v,
                us0, us1, ms0, ms1):
        wid = lax.axis_index("s") * _NC + lax.axis_index("c")
        row0 = wid * chunks
        base = wid * b_per_w
        usems = (us0, us1)
        msems = (ms0, ms1)

        pltpu.sync_copy(uidx_hbm.at[pl.ds(row0, chunks)], uidx_v)
        pltpu.sync_copy(midx_hbm.at[pl.ds(row0, chunks)], midx_v)

        def fire(j):
            return (
                pltpu.async_copy(utab_hbm.at[uidx_v.at[j]],
                                 ubuf.at[j % 2], usems[j % 2]),
                pltpu.async_copy(mtab_hbm.at[midx_v.at[j]],
                                 mbuf.at[j % 2], msems[j % 2]),
            )

        lane = lax.iota(jnp.int32, 16)
        handles = [None] * chunks
        for j in range(min(2, chunks)):
            handles[j] = fire(j)
        for j in range(chunks):
            hu, hm = handles[j]
            hu.wait()
            hm.wait()
            ub = ubuf.at[j % 2]
            mb = mbuf.at[j % 2]

            def compute(g, _):
                rows = g * 16 + lane
                col0 = jnp.zeros((16,), jnp.int32)
                acc = (plsc.load_gather(ub, [rows, col0])
                       * plsc.load_gather(mb, [rows, col0]))
                for o in range(1, 10):
                    colo = col0 + o
                    acc = acc + (plsc.load_gather(ub, [rows, colo])
                                 * plsc.load_gather(mb, [rows, colo]))
                z_v[pl.ds(j * _CHUNK + g * 16, 16)] = acc
                return 0

            lax.fori_loop(0, _CHUNK // 16, compute, 0)
            if j + 2 < chunks:
                handles[j + 2] = fire(j + 2)
        pltpu.sync_copy(z_v, z_out.at[pl.ds(base, b_per_w)])

    return sc_zdot


def kernel(user_input, movie_input, user_emb, item_emb, Wu, bu, Wm, bm):
    B = user_input.shape[0]
    K = user_emb.shape[1]
    H = Wu.shape[1]

    wu = jnp.zeros((K, _HP), jnp.float32).at[:, :H].set(Wu)
    wm = jnp.zeros((K, _HP), jnp.float32).at[:, :H].set(Wm)
    bup = jnp.zeros((1, _HP), jnp.float32).at[0, :H].set(bu)
    bmp = jnp.zeros((1, _HP), jnp.float32).at[0, :H].set(bm)

    nu = -(-user_emb.shape[0] // _BLKN)
    nm = -(-item_emb.shape[0] // _BLKN)
    uproj = _project(user_emb.T, wu, bup, nu)
    mproj = _project(item_emb.T, wm, bmp, nm)

    uidx = user_input.reshape(B // _CHUNK, _CHUNK)
    midx = movie_input.reshape(B // _CHUNK, _CHUNK)

    z = _make_sc_zdot(B, H)(uidx, midx, uproj, mproj)
    return z.reshape(B, 1)


# speedup vs baseline: 1.6616x; 1.6616x over previous
import functools

import jax
import jax.numpy as jnp
from jax import lax
from jax.experimental import pallas as pl
from jax.experimental.pallas import tpu as pltpu
from jax.experimental.pallas import tpu_sc as plsc

_NC = 2
_NS = 16
_NW = _NC * _NS
_CHUNK = 128
_G = 8
_HP = 16
_BLKN = 4096


def _proj_body(t_ref, w_ref, b_ref, o_ref):
    p = lax.dot_general(w_ref[...], t_ref[...], (((0,), (0,)), ((), ())),
                        preferred_element_type=jnp.float32) + b_ref[...]
    pt = p.T.reshape(_BLKN // _G, _G, _HP)
    o_ref[...] = jnp.concatenate([pt[:, j, :] for j in range(_G)], axis=-1)


def _project(tab_t, w, b, nsteps):
    K = tab_t.shape[0]
    return pl.pallas_call(
        _proj_body,
        grid=(nsteps,),
        in_specs=[
            pl.BlockSpec((K, _BLKN), lambda i: (0, i)),
            pl.BlockSpec((K, _HP), lambda i: (0, 0)),
            pl.BlockSpec((_HP, 1), lambda i: (0, 0)),
        ],
        out_specs=pl.BlockSpec((_BLKN // _G, _G * _HP), lambda i: (i, 0)),
        out_shape=jax.ShapeDtypeStruct((nsteps * _BLKN // _G, _G * _HP),
                                       jnp.float32),
    )(tab_t, w, b)


def _make_sc_zdot(B, H):
    assert B % (_NW * _CHUNK) == 0
    b_per_w = B // _NW
    chunks = b_per_w // _CHUNK
    W = _G * _HP
    mesh = plsc.VectorSubcoreMesh(core_axis_name="c", subcore_axis_name="s")

    @functools.partial(
        pl.kernel,
        mesh=mesh,
        out_type=jax.ShapeDtypeStruct((B,), jnp.float32),
        scratch_types=[
            pltpu.VMEM((chunks, _CHUNK), jnp.int32),
            pltpu.VMEM((chunks, _CHUNK), jnp.int32),
            pltpu.VMEM((chunks, _CHUNK), jnp.int32),
            pltpu.VMEM((chunks, _CHUNK), jnp.int32),
            pltpu.VMEM((2, _CHUNK, W), jnp.float32),
            pltpu.VMEM((2, _CHUNK, W), jnp.float32),
            pltpu.VMEM((b_per_w,), jnp.float32),
            pltpu.SemaphoreType.DMA,
            pltpu.SemaphoreType.DMA,
            pltpu.SemaphoreType.DMA,
            pltpu.SemaphoreType.DMA,
        ],
        compiler_params=pltpu.CompilerParams(needs_layout_passes=False,
                                             use_tc_tiling_on_sc=True),
    )
    def sc_zdot(uhi_hbm, uoff_hbm, mhi_hbm, moff_hbm, utab_hbm, mtab_hbm,
                z_out, uhi_v, uoff_v, mhi_v, moff_v, ubuf, mbuf, z_v,
                us0, us1, ms0, ms1):
        wid = lax.axis_index("s") * _NC + lax.axis_index("c")
        row0 = wid * chunks
        base = wid * b_per_w
        usems = (us0, us1)
        msems = (ms0, ms1)

        pltpu.sync_copy(uhi_hbm.at[pl.ds(row0, chunks)], uhi_v)
        pltpu.sync_copy(uoff_hbm.at[pl.ds(row0, chunks)], uoff_v)
        pltpu.sync_copy(mhi_hbm.at[pl.ds(row0, chunks)], mhi_v)
        pltpu.sync_copy(moff_hbm.at[pl.ds(row0, chunks)], moff_v)

        def fire(j):
            return (
                pltpu.async_copy(utab_hbm.at[uhi_v.at[j]],
                                 ubuf.at[j % 2], usems[j % 2]),
                pltpu.async_copy(mtab_hbm.at[mhi_v.at[j]],
                                 mbuf.at[j % 2], msems[j % 2]),
            )

        lane = lax.iota(jnp.int32, 16)
        handles = [None] * chunks
        for j in range(min(2, chunks)):
            handles[j] = fire(j)
        for j in range(chunks):
            hu, hm = handles[j]
            hu.wait()
            hm.wait()
            ub = ubuf.at[j % 2]
            mb = mbuf.at[j % 2]

            def compute(g, _):
                rows = g * 16 + lane
                uo = uoff_v[j, pl.ds(g * 16, 16)]
                mo = moff_v[j, pl.ds(g * 16, 16)]
                acc = (plsc.load_gather(ub, [rows, uo])
                       * plsc.load_gather(mb, [rows, mo]))
                for o in range(1, 10):
                    acc = acc + (plsc.load_gather(ub, [rows, uo + o])
                                 * plsc.load_gather(mb, [rows, mo + o]))
                z_v[pl.ds(j * _CHUNK + g * 16, 16)] = acc
                return 0

            lax.fori_loop(0, _CHUNK // 16, compute, 0)
            if j + 2 < chunks:
                handles[j + 2] = fire(j + 2)
        pltpu.sync_copy(z_v, z_out.at[pl.ds(base, b_per_w)])

    return sc_zdot


def kernel(user_input, movie_input, user_emb, item_emb, Wu, bu, Wm, bm):
    B = user_input.shape[0]
    K = user_emb.shape[1]
    H = Wu.shape[1]

    wu = jnp.zeros((K, _HP), jnp.float32).at[:, :H].set(Wu)
    wm = jnp.zeros((K, _HP), jnp.float32).at[:, :H].set(Wm)
    bup = jnp.zeros((_HP, 1), jnp.float32).at[:H, 0].set(bu)
    bmp = jnp.zeros((_HP, 1), jnp.float32).at[:H, 0].set(bm)

    nu = -(-user_emb.shape[0] // _BLKN)
    nm = -(-item_emb.shape[0] // _BLKN)
    uproj = _project(user_emb.T, wu, bup, nu)
    mproj = _project(item_emb.T, wm, bmp, nm)

    uidx = user_input.reshape(B // _CHUNK, _CHUNK)
    midx = movie_input.reshape(B // _CHUNK, _CHUNK)
    uhi, uoff = uidx >> 3, (uidx & (_G - 1)) * _HP
    mhi, moff = midx >> 3, (midx & (_G - 1)) * _HP

    z = _make_sc_zdot(B, H)(uhi, uoff, mhi, moff, uproj, mproj)
    return z.reshape(B, 1)


# speedup vs baseline: 1.7307x; 1.0416x over previous
import functools

import jax
import jax.numpy as jnp
from jax import lax
from jax.experimental import pallas as pl
from jax.experimental.pallas import tpu as pltpu
from jax.experimental.pallas import tpu_sc as plsc

_NC = 2
_NS = 16
_NW = _NC * _NS
_CHUNK = 128
_G = 8
_HP = 16
_BLKN = 4096


def _proj_body(t_ref, w_ref, b_ref, o_ref):
    p = lax.dot_general(t_ref[...], w_ref[...], (((0,), (0,)), ((), ())),
                        preferred_element_type=jnp.float32) + b_ref[...]
    p3 = p.reshape(_BLKN // _G, _G, _HP)
    o_ref[...] = jnp.concatenate([p3[:, j, :] for j in range(_G)], axis=-1)


def _project(tab_t, w, b, nsteps):
    K = tab_t.shape[0]
    return pl.pallas_call(
        _proj_body,
        grid=(nsteps,),
        in_specs=[
            pl.BlockSpec((K, _BLKN), lambda i: (0, i)),
            pl.BlockSpec((K, _HP), lambda i: (0, 0)),
            pl.BlockSpec((1, _HP), lambda i: (0, 0)),
        ],
        out_specs=pl.BlockSpec((_BLKN // _G, _G * _HP), lambda i: (i, 0)),
        out_shape=jax.ShapeDtypeStruct((nsteps * _BLKN // _G, _G * _HP),
                                       jnp.float32),
    )(tab_t, w, b)


def _make_sc_zdot(B, H):
    assert B % (_NW * _CHUNK) == 0
    b_per_w = B // _NW
    chunks = b_per_w // _CHUNK
    W = _G * _HP
    mesh = plsc.VectorSubcoreMesh(core_axis_name="c", subcore_axis_name="s")

    @functools.partial(
        pl.kernel,
        mesh=mesh,
        out_type=jax.ShapeDtypeStruct((B,), jnp.float32),
        scratch_types=[
            pltpu.VMEM((chunks, _CHUNK), jnp.int32),
            pltpu.VMEM((chunks, _CHUNK), jnp.int32),
            pltpu.VMEM((chunks, _CHUNK), jnp.int32),
            pltpu.VMEM((chunks, _CHUNK), jnp.int32),
            pltpu.VMEM((2, _CHUNK, W), jnp.float32),
            pltpu.VMEM((2, _CHUNK, W), jnp.float32),
            pltpu.VMEM((b_per_w,), jnp.float32),
            pltpu.SemaphoreType.DMA,
            pltpu.SemaphoreType.DMA,
            pltpu.SemaphoreType.DMA,
            pltpu.SemaphoreType.DMA,
        ],
        compiler_params=pltpu.CompilerParams(needs_layout_passes=False,
                                             use_tc_tiling_on_sc=True),
    )
    def sc_zdot(uhi_hbm, uoff_hbm, mhi_hbm, moff_hbm, utab_hbm, mtab_hbm,
                z_out, uhi_v, uoff_v, mhi_v, moff_v, ubuf, mbuf, z_v,
                us0, us1, ms0, ms1):
        wid = lax.axis_index("s") * _NC + lax.axis_index("c")
        row0 = wid * chunks
        base = wid * b_per_w
        usems = (us0, us1)
        msems = (ms0, ms1)

        pltpu.sync_copy(uhi_hbm.at[pl.ds(row0, chunks)], uhi_v)
        pltpu.sync_copy(uoff_hbm.at[pl.ds(row0, chunks)], uoff_v)
        pltpu.sync_copy(mhi_hbm.at[pl.ds(row0, chunks)], mhi_v)
        pltpu.sync_copy(moff_hbm.at[pl.ds(row0, chunks)], moff_v)

        def fire(j):
            return (
                pltpu.async_copy(utab_hbm.at[uhi_v.at[j]],
                                 ubuf.at[j % 2], usems[j % 2]),
                pltpu.async_copy(mtab_hbm.at[mhi_v.at[j]],
                                 mbuf.at[j % 2], msems[j % 2]),
            )

        lane = lax.iota(jnp.int32, 16)
        handles = [None] * chunks
        for j in range(min(2, chunks)):
            handles[j] = fire(j)
        for j in range(chunks):
            hu, hm = handles[j]
            hu.wait()
            hm.wait()
            ub = ubuf.at[j % 2]
            mb = mbuf.at[j % 2]

            def compute(g, _):
                rows = g * 16 + lane
                uo = uoff_v[j, pl.ds(g * 16, 16)]
                mo = moff_v[j, pl.ds(g * 16, 16)]
                acc = (plsc.load_gather(ub, [rows, uo])
                       * plsc.load_gather(mb, [rows, mo]))
                for o in range(1, 10):
                    acc = acc + (plsc.load_gather(ub, [rows, uo + o])
                                 * plsc.load_gather(mb, [rows, mo + o]))
                z_v[pl.ds(j * _CHUNK + g * 16, 16)] = acc
                return 0

            lax.fori_loop(0, _CHUNK // 16, compute, 0)
            if j + 2 < chunks:
                handles[j + 2] = fire(j + 2)
        pltpu.sync_copy(z_v, z_out.at[pl.ds(base, b_per_w)])

    return sc_zdot


def kernel(user_input, movie_input, user_emb, item_emb, Wu, bu, Wm, bm):
    B = user_input.shape[0]
    K = user_emb.shape[1]
    H = Wu.shape[1]

    wu = jnp.zeros((K, _HP), jnp.float32).at[:, :H].set(Wu)
    wm = jnp.zeros((K, _HP), jnp.float32).at[:, :H].set(Wm)
    bup = jnp.zeros((1, _HP), jnp.float32).at[0, :H].set(bu)
    bmp = jnp.zeros((1, _HP), jnp.float32).at[0, :H].set(bm)

    nu = -(-user_emb.shape[0] // _BLKN)
    nm = -(-item_emb.shape[0] // _BLKN)
    uproj = _project(user_emb.T, wu, bup, nu)
    mproj = _project(item_emb.T, wm, bmp, nm)

    uidx = user_input.reshape(B // _CHUNK, _CHUNK)
    midx = movie_input.reshape(B // _CHUNK, _CHUNK)
    uhi, uoff = uidx >> 3, (uidx & (_G - 1)) * _HP
    mhi, moff = midx >> 3, (midx & (_G - 1)) * _HP

    z = _make_sc_zdot(B, H)(uhi, uoff, mhi, moff, uproj, mproj)
    return z.reshape(B, 1)


# speedup vs baseline: 2.2798x; 1.3173x over previous
import functools

import jax
import jax.numpy as jnp
from jax import lax
from jax.experimental import pallas as pl
from jax.experimental.pallas import tpu as pltpu
from jax.experimental.pallas import tpu_sc as plsc

_NC = 2
_NS = 16
_NW = _NC * _NS
_CHUNK = 32
_HP = 16
_BLKN = 4096
_LT = _BLKN // 128


def _proj_body(t_ref, w_ref, b_ref, main_ref, a8_ref, a9_ref):
    p = lax.dot_general(w_ref[...], t_ref[...], (((0,), (0,)), ((), ())),
                        preferred_element_type=jnp.float32) + b_ref[...]
    for st in range(2):
        for lt in range(_LT):
            main_ref[pl.ds((st * _LT + lt) * 8, 8), :] = (
                p[st * 8:st * 8 + 8, lt * 128:lt * 128 + 128])
    for lt in range(_LT):
        a8_ref[pl.ds(lt, 1), :] = p[8:9, lt * 128:lt * 128 + 128]
        a9_ref[pl.ds(lt, 1), :] = p[9:10, lt * 128:lt * 128 + 128]


def _project(tab_t, w, b, nsteps):
    K = tab_t.shape[0]
    return pl.pallas_call(
        _proj_body,
        grid=(nsteps,),
        in_specs=[
            pl.BlockSpec((K, _BLKN), lambda i: (0, i)),
            pl.BlockSpec((K, _HP), lambda i: (0, 0)),
            pl.BlockSpec((_HP, 1), lambda i: (0, 0)),
        ],
        out_specs=[
            pl.BlockSpec((2 * _LT * 8, 128), lambda i: (i, 0)),
            pl.BlockSpec((_LT, 128), lambda i: (i, 0)),
            pl.BlockSpec((_LT, 128), lambda i: (i, 0)),
        ],
        out_shape=[
            jax.ShapeDtypeStruct((nsteps * 2 * _LT * 8, 128), jnp.float32),
            jax.ShapeDtypeStruct((nsteps * _LT, 128), jnp.float32),
            jax.ShapeDtypeStruct((nsteps * _LT, 128), jnp.float32),
        ],
    )(tab_t, w, b)


def _make_sc_zdot(B):
    assert B % (_NW * _CHUNK) == 0
    b_per_w = B // _NW
    chunks = b_per_w // _CHUNK
    mesh = plsc.VectorSubcoreMesh(core_axis_name="c", subcore_axis_name="s")

    @functools.partial(
        pl.kernel,
        mesh=mesh,
        out_type=jax.ShapeDtypeStruct((B,), jnp.float32),
        scratch_types=[
            pltpu.VMEM((chunks, 8, _CHUNK), jnp.int32),
            pltpu.VMEM((chunks, 8, _CHUNK), jnp.int32),
            pltpu.VMEM((chunks, _CHUNK), jnp.int32),
            pltpu.VMEM((chunks, _CHUNK), jnp.int32),
            pltpu.VMEM((chunks, _CHUNK), jnp.int32),
            pltpu.VMEM((chunks, _CHUNK), jnp.int32),
            pltpu.VMEM((8, _CHUNK, 128), jnp.float32),
            pltpu.VMEM((8, _CHUNK, 128), jnp.float32),
            pltpu.VMEM((2, _CHUNK, 128), jnp.float32),
            pltpu.VMEM((2, _CHUNK, 128), jnp.float32),
            pltpu.VMEM((b_per_w,), jnp.float32),
            pltpu.SemaphoreType.DMA,
        ],
        compiler_params=pltpu.CompilerParams(needs_layout_passes=False,
                                             use_tc_tiling_on_sc=True),
    )
    def sc_zdot(uidx_hbm, midx_hbm, uaux_hbm, maux_hbm, ula_hbm, mla_hbm,
                umain_hbm, ua8_hbm, ua9_hbm, mmain_hbm, ma8_hbm, ma9_hbm,
                z_out, uidx_v, midx_v, uaux_v, maux_v, ula_v, mla_v,
                ubuf, mbuf, uabuf, mabuf, z_v, sem):
        wid = lax.axis_index("s") * _NC + lax.axis_index("c")
        row0 = wid * chunks
        base = wid * b_per_w

        pltpu.sync_copy(uidx_hbm.at[pl.ds(row0, chunks)], uidx_v)
        pltpu.sync_copy(midx_hbm.at[pl.ds(row0, chunks)], midx_v)
        pltpu.sync_copy(uaux_hbm.at[pl.ds(row0, chunks)], uaux_v)
        pltpu.sync_copy(maux_hbm.at[pl.ds(row0, chunks)], maux_v)
        pltpu.sync_copy(ula_hbm.at[pl.ds(row0, chunks)], ula_v)
        pltpu.sync_copy(mla_hbm.at[pl.ds(row0, chunks)], mla_v)

        lane = lax.iota(jnp.int32, 16)
        for c in range(chunks):
            handles = []
            for su in range(8):
                handles.append(pltpu.async_copy(
                    umain_hbm.at[uidx_v.at[c, su]], ubuf.at[su], sem))
                handles.append(pltpu.async_copy(
                    mmain_hbm.at[midx_v.at[c, su]], mbuf.at[su], sem))
            handles.append(pltpu.async_copy(
                ua8_hbm.at[uaux_v.at[c]], uabuf.at[0], sem))
            handles.append(pltpu.async_copy(
                ua9_hbm.at[uaux_v.at[c]], uabuf.at[1], sem))
            handles.append(pltpu.async_copy(
                ma8_hbm.at[maux_v.at[c]], mabuf.at[0], sem))
            handles.append(pltpu.async_copy(
                ma9_hbm.at[maux_v.at[c]], mabuf.at[1], sem))
            for h in handles:
                h.wait()

            for g in range(_CHUNK // 16):
                rows = g * 16 + lane
                la_u = ula_v[c, pl.ds(g * 16, 16)]
                la_m = mla_v[c, pl.ds(g * 16, 16)]
                acc = None
                for o in range(8):
                    ov = jnp.full((16,), o, jnp.int32)
                    term = (plsc.load_gather(ubuf, [ov, rows, la_u])
                            * plsc.load_gather(mbuf, [ov, rows, la_m]))
                    acc = term if acc is None else acc + term
                for a in range(2):
                    av = jnp.full((16,), a, jnp.int32)
                    acc = acc + (plsc.load_gather(uabuf, [av, rows, la_u])
                                 * plsc.load_gather(mabuf, [av, rows, la_m]))
                z_v[pl.ds(c * _CHUNK + g * 16, 16)] = acc
        pltpu.sync_copy(z_v, z_out.at[pl.ds(base, b_per_w)])

    return sc_zdot


def _sc_indices(idx, B):
    i = idx >> 12
    lt = (idx & 4095) >> 7
    la = idx & 127
    basem = i * (2 * _LT * 8) + lt * 8
    main3 = (basem.reshape(B // _CHUNK, 1, _CHUNK)
             + jnp.arange(8, dtype=jnp.int32).reshape(1, 8, 1))
    aux = (i * _LT + lt).reshape(B // _CHUNK, _CHUNK)
    return main3, aux, la.reshape(B // _CHUNK, _CHUNK)


def kernel(user_input, movie_input, user_emb, item_emb, Wu, bu, Wm, bm):
    B = user_input.shape[0]
    K = user_emb.shape[1]
    H = Wu.shape[1]

    wu = jnp.zeros((K, _HP), jnp.float32).at[:, :H].set(Wu)
    wm = jnp.zeros((K, _HP), jnp.float32).at[:, :H].set(Wm)
    bup = jnp.zeros((_HP, 1), jnp.float32).at[:H, 0].set(bu)
    bmp = jnp.zeros((_HP, 1), jnp.float32).at[:H, 0].set(bm)

    nu = -(-user_emb.shape[0] // _BLKN)
    nm = -(-item_emb.shape[0] // _BLKN)
    umain, ua8, ua9 = _project(user_emb.T, wu, bup, nu)
    mmain, ma8, ma9 = _project(item_emb.T, wm, bmp, nm)

    uidx3, uaux, ula = _sc_indices(user_input.reshape(B), B)
    midx3, maux, mla = _sc_indices(movie_input.reshape(B), B)

    z = _make_sc_zdot(B)(uidx3, midx3, uaux, maux, ula, mla,
                         umain, ua8, ua9, mmain, ma8, ma9)
    return z.reshape(B, 1)


# speedup vs baseline: 2.9585x; 1.2977x over previous
import functools

import jax
import jax.numpy as jnp
from jax import lax
from jax.experimental import pallas as pl
from jax.experimental.pallas import tpu as pltpu
from jax.experimental.pallas import tpu_sc as plsc

_NC = 2
_NS = 16
_NW = _NC * _NS
_CHUNK = 32
_HP = 16
_BLKN = 8192
_LT = _BLKN // 128


def _proj_body(t_ref, w_ref, b_ref, main_ref, a8_ref, a9_ref):
    p = lax.dot_general(w_ref[...], t_ref[...], (((0,), (0,)), ((), ())),
                        preferred_element_type=jnp.float32) + b_ref[...]
    for st in range(2):
        for lt in range(_LT):
            main_ref[pl.ds((st * _LT + lt) * 8, 8), :] = (
                p[st * 8:st * 8 + 8, lt * 128:lt * 128 + 128])
    for lt in range(_LT):
        a8_ref[pl.ds(lt, 1), :] = p[8:9, lt * 128:lt * 128 + 128]
        a9_ref[pl.ds(lt, 1), :] = p[9:10, lt * 128:lt * 128 + 128]


def _project(tab_t, w, b, nsteps):
    K = tab_t.shape[0]
    return pl.pallas_call(
        _proj_body,
        grid=(nsteps,),
        in_specs=[
            pl.BlockSpec((K, _BLKN), lambda i: (0, i)),
            pl.BlockSpec((K, _HP), lambda i: (0, 0)),
            pl.BlockSpec((_HP, 1), lambda i: (0, 0)),
        ],
        out_specs=[
            pl.BlockSpec((2 * _LT * 8, 128), lambda i: (i, 0)),
            pl.BlockSpec((_LT, 128), lambda i: (i, 0)),
            pl.BlockSpec((_LT, 128), lambda i: (i, 0)),
        ],
        out_shape=[
            jax.ShapeDtypeStruct((nsteps * 2 * _LT * 8, 128), jnp.float32),
            jax.ShapeDtypeStruct((nsteps * _LT, 128), jnp.float32),
            jax.ShapeDtypeStruct((nsteps * _LT, 128), jnp.float32),
        ],
    )(tab_t, w, b)


def _make_sc_zdot(B):
    assert B % (_NW * _CHUNK) == 0
    b_per_w = B // _NW
    chunks = b_per_w // _CHUNK
    mesh = plsc.VectorSubcoreMesh(core_axis_name="c", subcore_axis_name="s")

    @functools.partial(
        pl.kernel,
        mesh=mesh,
        out_type=jax.ShapeDtypeStruct((B,), jnp.float32),
        scratch_types=[
            pltpu.VMEM((chunks, 8, _CHUNK), jnp.int32),
            pltpu.VMEM((chunks, 8, _CHUNK), jnp.int32),
            pltpu.VMEM((chunks, _CHUNK), jnp.int32),
            pltpu.VMEM((chunks, _CHUNK), jnp.int32),
            pltpu.VMEM((chunks, _CHUNK), jnp.int32),
            pltpu.VMEM((chunks, _CHUNK), jnp.int32),
            pltpu.VMEM((8, _CHUNK, 128), jnp.float32),
            pltpu.VMEM((8, _CHUNK, 128), jnp.float32),
            pltpu.VMEM((2, _CHUNK, 128), jnp.float32),
            pltpu.VMEM((2, _CHUNK, 128), jnp.float32),
            pltpu.VMEM((b_per_w,), jnp.float32),
            pltpu.SemaphoreType.DMA,
        ],
        compiler_params=pltpu.CompilerParams(needs_layout_passes=False,
                                             use_tc_tiling_on_sc=True),
    )
    def sc_zdot(uidx_hbm, midx_hbm, uaux_hbm, maux_hbm, ula_hbm, mla_hbm,
                umain_hbm, ua8_hbm, ua9_hbm, mmain_hbm, ma8_hbm, ma9_hbm,
                z_out, uidx_v, midx_v, uaux_v, maux_v, ula_v, mla_v,
                ubuf, mbuf, uabuf, mabuf, z_v, sem):
        wid = lax.axis_index("s") * _NC + lax.axis_index("c")
        row0 = wid * chunks
        base = wid * b_per_w

        pltpu.sync_copy(uidx_hbm.at[pl.ds(row0, chunks)], uidx_v)
        pltpu.sync_copy(midx_hbm.at[pl.ds(row0, chunks)], midx_v)
        pltpu.sync_copy(uaux_hbm.at[pl.ds(row0, chunks)], uaux_v)
        pltpu.sync_copy(maux_hbm.at[pl.ds(row0, chunks)], maux_v)
        pltpu.sync_copy(ula_hbm.at[pl.ds(row0, chunks)], ula_v)
        pltpu.sync_copy(mla_hbm.at[pl.ds(row0, chunks)], mla_v)

        lane = lax.iota(jnp.int32, 16)
        for c in range(chunks):
            handles = []
            for su in range(8):
                handles.append(pltpu.async_copy(
                    umain_hbm.at[uidx_v.at[c, su]], ubuf.at[su], sem))
                handles.append(pltpu.async_copy(
                    mmain_hbm.at[midx_v.at[c, su]], mbuf.at[su], sem))
            handles.append(pltpu.async_copy(
                ua8_hbm.at[uaux_v.at[c]], uabuf.at[0], sem))
            handles.append(pltpu.async_copy(
                ua9_hbm.at[uaux_v.at[c]], uabuf.at[1], sem))
            handles.append(pltpu.async_copy(
                ma8_hbm.at[maux_v.at[c]], mabuf.at[0], sem))
            handles.append(pltpu.async_copy(
                ma9_hbm.at[maux_v.at[c]], mabuf.at[1], sem))
            for h in handles:
                h.wait()

            for g in range(_CHUNK // 16):
                rows = g * 16 + lane
                la_u = ula_v[c, pl.ds(g * 16, 16)]
                la_m = mla_v[c, pl.ds(g * 16, 16)]
                acc = None
                for o in range(8):
                    ov = jnp.full((16,), o, jnp.int32)
                    term = (plsc.load_gather(ubuf, [ov, rows, la_u])
                            * plsc.load_gather(mbuf, [ov, rows, la_m]))
                    acc = term if acc is None else acc + term
                for a in range(2):
                    av = jnp.full((16,), a, jnp.int32)
                    acc = acc + (plsc.load_gather(uabuf, [av, rows, la_u])
                                 * plsc.load_gather(mabuf, [av, rows, la_m]))
                z_v[pl.ds(c * _CHUNK + g * 16, 16)] = acc
        pltpu.sync_copy(z_v, z_out.at[pl.ds(base, b_per_w)])

    return sc_zdot


def _sc_indices(idx, B):
    i = idx // _BLKN
    lt = (idx & (_BLKN - 1)) >> 7
    la = idx & 127
    basem = i * (2 * _LT * 8) + lt * 8
    main3 = (basem.reshape(B // _CHUNK, 1, _CHUNK)
             + jnp.arange(8, dtype=jnp.int32).reshape(1, 8, 1))
    aux = (i * _LT + lt).reshape(B // _CHUNK, _CHUNK)
    return main3, aux, la.reshape(B // _CHUNK, _CHUNK)


def kernel(user_input, movie_input, user_emb, item_emb, Wu, bu, Wm, bm):
    B = user_input.shape[0]
    K = user_emb.shape[1]
    H = Wu.shape[1]

    wu = jnp.zeros((K, _HP), jnp.float32).at[:, :H].set(Wu)
    wm = jnp.zeros((K, _HP), jnp.float32).at[:, :H].set(Wm)
    bup = jnp.zeros((_HP, 1), jnp.float32).at[:H, 0].set(bu)
    bmp = jnp.zeros((_HP, 1), jnp.float32).at[:H, 0].set(bm)

    nu = -(-user_emb.shape[0] // _BLKN)
    nm = -(-item_emb.shape[0] // _BLKN)
    umain, ua8, ua9 = _project(user_emb.T, wu, bup, nu)
    mmain, ma8, ma9 = _project(item_emb.T, wm, bmp, nm)

    uidx3, uaux, ula = _sc_indices(user_input.reshape(B), B)
    midx3, maux, mla = _sc_indices(movie_input.reshape(B), B)

    z = _make_sc_zdot(B)(uidx3, midx3, uaux, maux, ula, mla,
                         umain, ua8, ua9, mmain, ma8, ma9)
    return z.reshape(B, 1)


# speedup vs baseline: 3.5693x; 1.2065x over previous
import functools

import jax
import jax.numpy as jnp
from jax import lax
from jax.experimental import pallas as pl
from jax.experimental.pallas import tpu as pltpu
from jax.experimental.pallas import tpu_sc as plsc

_NC = 2
_NS = 16
_NW = _NC * _NS
_CHUNK = 32
_HP = 16
_BLKN = 16384
_LT = _BLKN // 128


def _proj_body(t_ref, w_ref, b_ref, main_ref, a8_ref, a9_ref):
    p = lax.dot_general(w_ref[...], t_ref[...], (((0,), (0,)), ((), ())),
                        preferred_element_type=jnp.float32) + b_ref[...]
    for st in range(2):
        for lt in range(_LT):
            main_ref[pl.ds((st * _LT + lt) * 8, 8), :] = (
                p[st * 8:st * 8 + 8, lt * 128:lt * 128 + 128])
    for lt in range(_LT):
        a8_ref[pl.ds(lt, 1), :] = p[8:9, lt * 128:lt * 128 + 128]
        a9_ref[pl.ds(lt, 1), :] = p[9:10, lt * 128:lt * 128 + 128]


def _project(tab_t, w, b, nsteps):
    K = tab_t.shape[0]
    return pl.pallas_call(
        _proj_body,
        grid=(nsteps,),
        in_specs=[
            pl.BlockSpec((K, _BLKN), lambda i: (0, i)),
            pl.BlockSpec((K, _HP), lambda i: (0, 0)),
            pl.BlockSpec((_HP, 1), lambda i: (0, 0)),
        ],
        out_specs=[
            pl.BlockSpec((2 * _LT * 8, 128), lambda i: (i, 0)),
            pl.BlockSpec((_LT, 128), lambda i: (i, 0)),
            pl.BlockSpec((_LT, 128), lambda i: (i, 0)),
        ],
        out_shape=[
            jax.ShapeDtypeStruct((nsteps * 2 * _LT * 8, 128), jnp.float32),
            jax.ShapeDtypeStruct((nsteps * _LT, 128), jnp.float32),
            jax.ShapeDtypeStruct((nsteps * _LT, 128), jnp.float32),
        ],
    )(tab_t, w, b)


def _make_sc_zdot(B):
    assert B % (_NW * _CHUNK) == 0
    b_per_w = B // _NW
    chunks = b_per_w // _CHUNK
    mesh = plsc.VectorSubcoreMesh(core_axis_name="c", subcore_axis_name="s")

    @functools.partial(
        pl.kernel,
        mesh=mesh,
        out_type=jax.ShapeDtypeStruct((B,), jnp.float32),
        scratch_types=[
            pltpu.VMEM((chunks, 8, _CHUNK), jnp.int32),
            pltpu.VMEM((chunks, 8, _CHUNK), jnp.int32),
            pltpu.VMEM((chunks, _CHUNK), jnp.int32),
            pltpu.VMEM((chunks, _CHUNK), jnp.int32),
            pltpu.VMEM((chunks, _CHUNK), jnp.int32),
            pltpu.VMEM((chunks, _CHUNK), jnp.int32),
            pltpu.VMEM((8, _CHUNK, 128), jnp.float32),
            pltpu.VMEM((8, _CHUNK, 128), jnp.float32),
            pltpu.VMEM((2, _CHUNK, 128), jnp.float32),
            pltpu.VMEM((2, _CHUNK, 128), jnp.float32),
            pltpu.VMEM((b_per_w,), jnp.float32),
            pltpu.SemaphoreType.DMA,
        ],
        compiler_params=pltpu.CompilerParams(needs_layout_passes=False,
                                             use_tc_tiling_on_sc=True),
    )
    def sc_zdot(uidx_hbm, midx_hbm, uaux_hbm, maux_hbm, ula_hbm, mla_hbm,
                umain_hbm, ua8_hbm, ua9_hbm, mmain_hbm, ma8_hbm, ma9_hbm,
                z_out, uidx_v, midx_v, uaux_v, maux_v, ula_v, mla_v,
                ubuf, mbuf, uabuf, mabuf, z_v, sem):
        wid = lax.axis_index("s") * _NC + lax.axis_index("c")
        row0 = wid * chunks
        base = wid * b_per_w

        pltpu.sync_copy(uidx_hbm.at[pl.ds(row0, chunks)], uidx_v)
        pltpu.sync_copy(midx_hbm.at[pl.ds(row0, chunks)], midx_v)
        pltpu.sync_copy(uaux_hbm.at[pl.ds(row0, chunks)], uaux_v)
        pltpu.sync_copy(maux_hbm.at[pl.ds(row0, chunks)], maux_v)
        pltpu.sync_copy(ula_hbm.at[pl.ds(row0, chunks)], ula_v)
        pltpu.sync_copy(mla_hbm.at[pl.ds(row0, chunks)], mla_v)

        lane = lax.iota(jnp.int32, 16)
        for c in range(chunks):
            handles = []
            for su in range(8):
                handles.append(pltpu.async_copy(
                    umain_hbm.at[uidx_v.at[c, su]], ubuf.at[su], sem))
                handles.append(pltpu.async_copy(
                    mmain_hbm.at[midx_v.at[c, su]], mbuf.at[su], sem))
            handles.append(pltpu.async_copy(
                ua8_hbm.at[uaux_v.at[c]], uabuf.at[0], sem))
            handles.append(pltpu.async_copy(
                ua9_hbm.at[uaux_v.at[c]], uabuf.at[1], sem))
            handles.append(pltpu.async_copy(
                ma8_hbm.at[maux_v.at[c]], mabuf.at[0], sem))
            handles.append(pltpu.async_copy(
                ma9_hbm.at[maux_v.at[c]], mabuf.at[1], sem))
            for h in handles:
                h.wait()

            for g in range(_CHUNK // 16):
                rows = g * 16 + lane
                la_u = ula_v[c, pl.ds(g * 16, 16)]
                la_m = mla_v[c, pl.ds(g * 16, 16)]
                acc = None
                for o in range(8):
                    ov = jnp.full((16,), o, jnp.int32)
                    term = (plsc.load_gather(ubuf, [ov, rows, la_u])
                            * plsc.load_gather(mbuf, [ov, rows, la_m]))
                    acc = term if acc is None else acc + term
                for a in range(2):
                    av = jnp.full((16,), a, jnp.int32)
                    acc = acc + (plsc.load_gather(uabuf, [av, rows, la_u])
                                 * plsc.load_gather(mabuf, [av, rows, la_m]))
                z_v[pl.ds(c * _CHUNK + g * 16, 16)] = acc
        pltpu.sync_copy(z_v, z_out.at[pl.ds(base, b_per_w)])

    return sc_zdot


def _sc_indices(idx, B):
    i = idx // _BLKN
    lt = (idx & (_BLKN - 1)) >> 7
    la = idx & 127
    basem = i * (2 * _LT * 8) + lt * 8
    main3 = (basem.reshape(B // _CHUNK, 1, _CHUNK)
             + jnp.arange(8, dtype=jnp.int32).reshape(1, 8, 1))
    aux = (i * _LT + lt).reshape(B // _CHUNK, _CHUNK)
    return main3, aux, la.reshape(B // _CHUNK, _CHUNK)


def kernel(user_input, movie_input, user_emb, item_emb, Wu, bu, Wm, bm):
    B = user_input.shape[0]
    K = user_emb.shape[1]
    H = Wu.shape[1]

    wu = jnp.zeros((K, _HP), jnp.float32).at[:, :H].set(Wu)
    wm = jnp.zeros((K, _HP), jnp.float32).at[:, :H].set(Wm)
    bup = jnp.zeros((_HP, 1), jnp.float32).at[:H, 0].set(bu)
    bmp = jnp.zeros((_HP, 1), jnp.float32).at[:H, 0].set(bm)

    nu = -(-user_emb.shape[0] // _BLKN)
    nm = -(-item_emb.shape[0] // _BLKN)
    umain, ua8, ua9 = _project(user_emb.T, wu, bup, nu)
    mmain, ma8, ma9 = _project(item_emb.T, wm, bmp, nm)

    uidx3, uaux, ula = _sc_indices(user_input.reshape(B), B)
    midx3, maux, mla = _sc_indices(movie_input.reshape(B), B)

    z = _make_sc_zdot(B)(uidx3, midx3, uaux, maux, ula, mla,
                         umain, ua8, ua9, mmain, ma8, ma9)
    return z.reshape(B, 1)


# speedup vs baseline: 3.8874x; 1.0891x over previous
import functools

import jax
import jax.numpy as jnp
from jax import lax
from jax.experimental import pallas as pl
from jax.experimental.pallas import tpu as pltpu
from jax.experimental.pallas import tpu_sc as plsc

_NC = 2
_NS = 16
_NW = _NC * _NS
_CHUNK = 32
_HP = 16
_BLKN = 32768
_LT = _BLKN // 128


def _proj_body(t_ref, w_ref, b_ref, main_ref, a8_ref, a9_ref):
    p = lax.dot_general(w_ref[...], t_ref[...], (((0,), (0,)), ((), ())),
                        preferred_element_type=jnp.float32) + b_ref[...]
    for st in range(2):
        for lt in range(_LT):
            main_ref[pl.ds((st * _LT + lt) * 8, 8), :] = (
                p[st * 8:st * 8 + 8, lt * 128:lt * 128 + 128])
    for lt in range(_LT):
        a8_ref[pl.ds(lt, 1), :] = p[8:9, lt * 128:lt * 128 + 128]
        a9_ref[pl.ds(lt, 1), :] = p[9:10, lt * 128:lt * 128 + 128]


def _project(tab_t, w, b, nsteps):
    K = tab_t.shape[0]
    return pl.pallas_call(
        _proj_body,
        grid=(nsteps,),
        in_specs=[
            pl.BlockSpec((K, _BLKN), lambda i: (0, i)),
            pl.BlockSpec((K, _HP), lambda i: (0, 0)),
            pl.BlockSpec((_HP, 1), lambda i: (0, 0)),
        ],
        out_specs=[
            pl.BlockSpec((2 * _LT * 8, 128), lambda i: (i, 0)),
            pl.BlockSpec((_LT, 128), lambda i: (i, 0)),
            pl.BlockSpec((_LT, 128), lambda i: (i, 0)),
        ],
        out_shape=[
            jax.ShapeDtypeStruct((nsteps * 2 * _LT * 8, 128), jnp.float32),
            jax.ShapeDtypeStruct((nsteps * _LT, 128), jnp.float32),
            jax.ShapeDtypeStruct((nsteps * _LT, 128), jnp.float32),
        ],
    )(tab_t, w, b)


def _make_sc_zdot(B):
    assert B % (_NW * _CHUNK) == 0
    b_per_w = B // _NW
    chunks = b_per_w // _CHUNK
    mesh = plsc.VectorSubcoreMesh(core_axis_name="c", subcore_axis_name="s")

    @functools.partial(
        pl.kernel,
        mesh=mesh,
        out_type=jax.ShapeDtypeStruct((B,), jnp.float32),
        scratch_types=[
            pltpu.VMEM((chunks, 8, _CHUNK), jnp.int32),
            pltpu.VMEM((chunks, 8, _CHUNK), jnp.int32),
            pltpu.VMEM((chunks, _CHUNK), jnp.int32),
            pltpu.VMEM((chunks, _CHUNK), jnp.int32),
            pltpu.VMEM((chunks, _CHUNK), jnp.int32),
            pltpu.VMEM((chunks, _CHUNK), jnp.int32),
            pltpu.VMEM((8, _CHUNK, 128), jnp.float32),
            pltpu.VMEM((8, _CHUNK, 128), jnp.float32),
            pltpu.VMEM((2, _CHUNK, 128), jnp.float32),
            pltpu.VMEM((2, _CHUNK, 128), jnp.float32),
            pltpu.VMEM((b_per_w,), jnp.float32),
            pltpu.SemaphoreType.DMA,
        ],
        compiler_params=pltpu.CompilerParams(needs_layout_passes=False,
                                             use_tc_tiling_on_sc=True),
    )
    def sc_zdot(uidx_hbm, midx_hbm, uaux_hbm, maux_hbm, ula_hbm, mla_hbm,
                umain_hbm, ua8_hbm, ua9_hbm, mmain_hbm, ma8_hbm, ma9_hbm,
                z_out, uidx_v, midx_v, uaux_v, maux_v, ula_v, mla_v,
                ubuf, mbuf, uabuf, mabuf, z_v, sem):
        wid = lax.axis_index("s") * _NC + lax.axis_index("c")
        row0 = wid * chunks
        base = wid * b_per_w

        pltpu.sync_copy(uidx_hbm.at[pl.ds(row0, chunks)], uidx_v)
        pltpu.sync_copy(midx_hbm.at[pl.ds(row0, chunks)], midx_v)
        pltpu.sync_copy(uaux_hbm.at[pl.ds(row0, chunks)], uaux_v)
        pltpu.sync_copy(maux_hbm.at[pl.ds(row0, chunks)], maux_v)
        pltpu.sync_copy(ula_hbm.at[pl.ds(row0, chunks)], ula_v)
        pltpu.sync_copy(mla_hbm.at[pl.ds(row0, chunks)], mla_v)

        lane = lax.iota(jnp.int32, 16)
        for c in range(chunks):
            handles = []
            for su in range(8):
                handles.append(pltpu.async_copy(
                    umain_hbm.at[uidx_v.at[c, su]], ubuf.at[su], sem))
                handles.append(pltpu.async_copy(
                    mmain_hbm.at[midx_v.at[c, su]], mbuf.at[su], sem))
            handles.append(pltpu.async_copy(
                ua8_hbm.at[uaux_v.at[c]], uabuf.at[0], sem))
            handles.append(pltpu.async_copy(
                ua9_hbm.at[uaux_v.at[c]], uabuf.at[1], sem))
            handles.append(pltpu.async_copy(
                ma8_hbm.at[maux_v.at[c]], mabuf.at[0], sem))
            handles.append(pltpu.async_copy(
                ma9_hbm.at[maux_v.at[c]], mabuf.at[1], sem))
            for h in handles:
                h.wait()

            for g in range(_CHUNK // 16):
                rows = g * 16 + lane
                la_u = ula_v[c, pl.ds(g * 16, 16)]
                la_m = mla_v[c, pl.ds(g * 16, 16)]
                acc = None
                for o in range(8):
                    ov = jnp.full((16,), o, jnp.int32)
                    term = (plsc.load_gather(ubuf, [ov, rows, la_u])
                            * plsc.load_gather(mbuf, [ov, rows, la_m]))
                    acc = term if acc is None else acc + term
                for a in range(2):
                    av = jnp.full((16,), a, jnp.int32)
                    acc = acc + (plsc.load_gather(uabuf, [av, rows, la_u])
                                 * plsc.load_gather(mabuf, [av, rows, la_m]))
                z_v[pl.ds(c * _CHUNK + g * 16, 16)] = acc
        pltpu.sync_copy(z_v, z_out.at[pl.ds(base, b_per_w)])

    return sc_zdot


def _sc_indices(idx, B):
    i = idx // _BLKN
    lt = (idx & (_BLKN - 1)) >> 7
    la = idx & 127
    basem = i * (2 * _LT * 8) + lt * 8
    main3 = (basem.reshape(B // _CHUNK, 1, _CHUNK)
             + jnp.arange(8, dtype=jnp.int32).reshape(1, 8, 1))
    aux = (i * _LT + lt).reshape(B // _CHUNK, _CHUNK)
    return main3, aux, la.reshape(B // _CHUNK, _CHUNK)


def kernel(user_input, movie_input, user_emb, item_emb, Wu, bu, Wm, bm):
    B = user_input.shape[0]
    K = user_emb.shape[1]
    H = Wu.shape[1]

    wu = jnp.zeros((K, _HP), jnp.float32).at[:, :H].set(Wu)
    wm = jnp.zeros((K, _HP), jnp.float32).at[:, :H].set(Wm)
    bup = jnp.zeros((_HP, 1), jnp.float32).at[:H, 0].set(bu)
    bmp = jnp.zeros((_HP, 1), jnp.float32).at[:H, 0].set(bm)

    nu = -(-user_emb.shape[0] // _BLKN)
    nm = -(-item_emb.shape[0] // _BLKN)
    umain, ua8, ua9 = _project(user_emb.T, wu, bup, nu)
    mmain, ma8, ma9 = _project(item_emb.T, wm, bmp, nm)

    uidx3, uaux, ula = _sc_indices(user_input.reshape(B), B)
    midx3, maux, mla = _sc_indices(movie_input.reshape(B), B)

    z = _make_sc_zdot(B)(uidx3, midx3, uaux, maux, ula, mla,
                         umain, ua8, ua9, mmain, ma8, ma9)
    return z.reshape(B, 1)
